# baseline (speedup 1.0000x reference)
import functools

import jax
import jax.numpy as jnp
from jax import lax
from jax.experimental import pallas as pl
from jax.experimental.pallas import tpu as pltpu
from jax.experimental.pallas import tpu_sc as plsc

N = 320000
B = 10000
D = 128
NC = 2
NS = 16
NW = NC * NS
C = 128
SS = 32
NSUB = C // SS
GPS = SS // 16
NCHUNK = N // C
TPW = -(-NCHUNK // NW)
BP = 10112
RPS = BP // NS
NRS = RPS // SS
SEGW = 16


def _sc_pass(h_react, batch2d, e2d, w1):
  mesh = plsc.VectorSubcoreMesh(core_axis_name="c", subcore_axis_name="s")

  @functools.partial(
      pl.kernel,
      out_type=(
          jax.ShapeDtypeStruct((NC * BP, D), jnp.float32),
          jax.ShapeDtypeStruct((NC * BP, SEGW), jnp.float32),
      ),
      mesh=mesh,
      compiler_params=pltpu.CompilerParams(needs_layout_passes=False),
      scratch_types=[
          pltpu.VMEM((SS, D), jnp.float32),
          pltpu.VMEM((SS, D), jnp.float32),
          pltpu.VMEM((NSUB, SS), jnp.int32),
          pltpu.VMEM((NSUB, SS), jnp.int32),
          pltpu.VMEM((1, SS), jnp.int32),
          pltpu.VMEM((SS, SEGW), jnp.float32),
          pltpu.VMEM((SS, SEGW), jnp.float32),
          pltpu.VMEM((D,), jnp.float32),
          pltpu.SemaphoreType.DMA,
          pltpu.SemaphoreType.DMA,
          pltpu.SemaphoreType.DMA,
          pltpu.SemaphoreType.DMA,
          pltpu.SemaphoreType.DMA,
          pltpu.SemaphoreType.DMA,
          pltpu.SemaphoreType.DMA,
          pltpu.SemaphoreType.DMA,
          pltpu.SemaphoreType.DMA,
          pltpu.SemaphoreType.DMA,
          pltpu.VMEM_SHARED((BP, D), jnp.float32),
          pltpu.VMEM_SHARED((BP, SEGW), jnp.float32),
      ],
  )
  def sc_kernel(h_hbm, b2d_hbm, e2d_hbm, w1_hbm,
                acc_out, seg_out,
                h_bufa, h_bufb, idx_bufa, idx_bufb, io_idx, ev_bufa, ev_bufb,
                w1_buf, sem_h0, sem_h1, sem_e0, sem_e1, sem_i0, sem_i1,
                sem_sa0, sem_sa1, sem_sb0, sem_sb1,
                acc_sh, es_sh):
    hb = [h_bufa, h_bufb]
    ixb = [idx_bufa, idx_bufb]
    evb = [ev_bufa, ev_bufb]
    sh = [sem_h0, sem_h1]
    se = [sem_e0, sem_e1]
    si = [sem_i0, sem_i1]
    ssa = [sem_sa0, sem_sa1]
    ssb = [sem_sb0, sem_sb1]
    cid = lax.axis_index("c")
    sid = lax.axis_index("s")
    wid = sid * NC + cid
    lanes = lax.iota(jnp.int32, 16)
    ecols = jnp.maximum(lanes, 1)
    zeros16i = jnp.zeros((16,), jnp.int32)
    zf16 = jnp.zeros((16,), jnp.float32)
    r0 = sid * RPS

    def set_io_idx(base):
      for c in range(SS // 16):
        io_idx[0, pl.ds(c * 16, 16)] = base + c * 16 + lanes

    def stream_base(rr):
      return r0 + jnp.minimum(rr * SS, RPS - SS)

    def _zrow(i, _):
      for kk in range(D // 16):
        h_bufa[i, pl.ds(kk * 16, 16)] = zf16
      return 0
    lax.fori_loop(0, SS, _zrow, 0)

    def _init(rr, _):
      base = stream_base(rr)
      set_io_idx(base)
      pltpu.sync_copy(h_bufa, acc_sh.at[io_idx.at[0]])
      pltpu.sync_copy(e2d_hbm.at[pl.ds(base, SS)], ev_bufa)
      pltpu.sync_copy(ev_bufa, es_sh.at[io_idx.at[0]])
      return 0
    lax.fori_loop(0, NRS + 1, _init, 0)
    pltpu.sync_copy(w1_hbm, w1_buf)
    w1v = [w1_buf[pl.ds(kk * 16, 16)] for kk in range(D // 16)]
    plsc.subcore_barrier()

    def issue_h(kc, jc, b):
      pltpu.async_copy(h_hbm.at[pl.ds(kc * C + jc * SS, SS)], hb[b], sh[b])

    def issue_ev(pc, jc, b):
      pltpu.async_copy(es_sh.at[ixb[pc].at[jc]], evb[b], se[b])

    def wait_h(b):
      pltpu.make_async_copy(h_hbm.at[pl.ds(0, SS)], hb[b], sh[b]).wait()

    def wait_ev(b):
      pltpu.make_async_copy(es_sh.at[ixb[0].at[0]], evb[b], se[b]).wait()

    def wait_scatters(b):
      pltpu.make_async_copy(hb[b], acc_sh.at[ixb[0].at[0]], ssa[b]).wait()
      pltpu.make_async_copy(evb[b], es_sh.at[ixb[0].at[0]], ssb[b]).wait()

    pltpu.async_copy(b2d_hbm.at[pl.ds(wid * NSUB, NSUB)], ixb[0], si[0])
    pltpu.make_async_copy(b2d_hbm.at[pl.ds(0, NSUB)], ixb[0], si[0]).wait()
    issue_h(wid, 0, 0)
    issue_ev(0, 0, 0)

    def compute_substage(cur, p, j):
      hc, ec = hb[cur], evb[cur]

      for g2 in range(GPS):
        gbase = g2 * 16
        e16 = plsc.load_gather(ec, (gbase + lanes, ecols))
        for jj in range(16):
          row = [hc[gbase + jj, pl.ds(kk * 16, 16)] for kk in range(D // 16)]
          prt = row[0] * w1v[0]
          for kk in range(1, D // 16):
            prt = prt + row[kk] * w1v[kk]
          zj = jnp.sum(prt) + e16[jj]
          zj = jnp.where(zj >= 0, zj, zj * 0.01)
          av = jnp.exp(jnp.full((16,), zj))
          for kk in range(D // 16):
            hc[gbase + jj, pl.ds(kk * 16, 16)] = row[kk] * av
          ec[gbase + jj, :] = jnp.where(lanes == 0, av, zf16)

    def pair_body(tp, _):
      for half in range(2):
        t = 2 * tp + half
        k = wid + t * NW
        p = half
        q = 1 - half

        @pl.when(k < NCHUNK)
        def _():
          for j in range(NSUB):
            cur = j % 2
            nxt = 1 - cur
            if half == 0 and j == 0:
              @pl.when(tp > 0)
              def _():
                wait_scatters(nxt)
            else:
              wait_scatters(nxt)
            if j == 0:
              @pl.when(k + NW < NCHUNK)
              def _():
                pltpu.async_copy(b2d_hbm.at[pl.ds((k + NW) * NSUB, NSUB)],
                                 ixb[q], si[q])
            if j < NSUB - 1:
              issue_h(k, j + 1, nxt)
              issue_ev(p, j + 1, nxt)
            else:
              @pl.when(k + NW < NCHUNK)
              def _():
                pltpu.make_async_copy(b2d_hbm.at[pl.ds(0, NSUB)],
                                      ixb[q], si[q]).wait()
                issue_h(k + NW, 0, nxt)
                issue_ev(q, 0, nxt)
            wait_h(cur)
            wait_ev(cur)
            compute_substage(cur, p, j)
            pltpu.async_copy(hb[cur], acc_sh.at[ixb[p].at[j]], ssa[cur])
            pltpu.async_copy(evb[cur], es_sh.at[ixb[p].at[j]], ssb[cur])
      return 0
    lax.fori_loop(0, (TPW + 1) // 2, pair_body, 0)
    wait_scatters(1)

    plsc.subcore_barrier()
    out_r0 = cid * BP + sid * RPS

    def _out(rr, _):
      base = stream_base(rr)
      set_io_idx(base)
      pltpu.sync_copy(acc_sh.at[io_idx.at[0]], h_bufa)
      pltpu.sync_copy(h_bufa, acc_out.at[pl.ds(out_r0 - r0 + base, SS)])
      pltpu.sync_copy(es_sh.at[io_idx.at[0]], ev_bufa)
      pltpu.sync_copy(ev_bufa, seg_out.at[pl.ds(out_r0 - r0 + base, SS)])
      return 0
    lax.fori_loop(0, NRS + 1, _out, 0)

  return sc_kernel(h_react, batch2d, e2d, w1)


def _e_kernel(henv_ref, w2_ref, b_ref, out_ref):
  ee = jnp.dot(henv_ref[...], w2_ref[...],
               preferred_element_type=jnp.float32) + b_ref[0, 0]
  col = lax.broadcasted_iota(jnp.int32, (B, SEGW), 1)
  out_ref[pl.ds(0, B), :] = jnp.where(col == 0, 0.0,
                                      jnp.broadcast_to(ee, (B, SEGW)))
  out_ref[pl.ds(B, BP - B), :] = jnp.zeros((BP - B, SEGW), jnp.float32)


def _mlp_kernel(acc_ref, seg_ref, henv_ref, fc1w_ref, fc1b_ref,
                fc2w_ref, fc2b_ref, out_ref):
  acc = acc_ref[pl.ds(0, B), :] + acc_ref[pl.ds(BP, B), :]
  seg = seg_ref[pl.ds(0, B), :] + seg_ref[pl.ds(BP, B), :]
  seg = seg[:, 0:1]
  seg = jnp.where(seg > 0, seg, 1.0)
  h_attn = acc / seg
  w_a = fc1w_ref[pl.ds(0, D), :]
  w_e = fc1w_ref[pl.ds(D, D), :]
  h1 = (jnp.dot(h_attn, w_a, preferred_element_type=jnp.float32)
        + jnp.dot(henv_ref[...], w_e, preferred_element_type=jnp.float32)
        + fc1b_ref[...])
  h1 = jnp.maximum(h1, 0.0)
  out_ref[...] = (jnp.dot(h1, fc2w_ref[...],
                          preferred_element_type=jnp.float32) + fc2b_ref[...])


def kernel(h_react, h_env, batch, attn_w, attn_b, fc1_w, fc1_b, fc2_w, fc2_b):
  w1 = attn_w[:D, 0]
  w2 = attn_w[D:, :]

  e2d = pl.pallas_call(
      _e_kernel,
      out_shape=jax.ShapeDtypeStruct((BP, SEGW), jnp.float32),
  )(h_env, w2, attn_b.reshape(1, 1))

  batch2d = batch.reshape(N // SS, SS)
  acc, seg = _sc_pass(h_react, batch2d, e2d, w1)

  out = pl.pallas_call(
      _mlp_kernel,
      out_shape=jax.ShapeDtypeStruct((B, 1), jnp.float32),
  )(acc, seg, h_env, fc1_w, fc1_b.reshape(1, 64), fc2_w, fc2_b.reshape(1, 1))
  return out


# speedup vs baseline: 1.3746x; 1.3746x over previous
import functools

import jax
import jax.numpy as jnp
from jax import lax
from jax.experimental import pallas as pl
from jax.experimental.pallas import tpu as pltpu
from jax.experimental.pallas import tpu_sc as plsc

N = 320000
B = 10000
D = 128
NC = 2
NS = 16
NW = NC * NS
C = 128
SS = 32
NSUB = C // SS
GPS = SS // 16
NCHUNK = N // C
TPW = -(-NCHUNK // NW)
BP = 10112
RPS = BP // NS
NRS = RPS // SS
SEGW = 16


def _sc_pass(h_react, batch2d, e2d, w1):
  mesh = plsc.VectorSubcoreMesh(core_axis_name="c", subcore_axis_name="s")

  @functools.partial(
      pl.kernel,
      out_type=(
          jax.ShapeDtypeStruct((NC * BP, D), jnp.float32),
          jax.ShapeDtypeStruct((NC * BP, SEGW), jnp.float32),
      ),
      mesh=mesh,
      compiler_params=pltpu.CompilerParams(needs_layout_passes=False),
      scratch_types=[
          pltpu.VMEM((SS, D), jnp.float32),
          pltpu.VMEM((SS, D), jnp.float32),
          pltpu.VMEM((NSUB, SS), jnp.int32),
          pltpu.VMEM((NSUB, SS), jnp.int32),
          pltpu.VMEM((1, SS), jnp.int32),
          pltpu.VMEM((SS, SEGW), jnp.float32),
          pltpu.VMEM((SS, SEGW), jnp.float32),
          pltpu.VMEM((D,), jnp.float32),
          pltpu.SemaphoreType.DMA,
          pltpu.SemaphoreType.DMA,
          pltpu.SemaphoreType.DMA,
          pltpu.SemaphoreType.DMA,
          pltpu.SemaphoreType.DMA,
          pltpu.SemaphoreType.DMA,
          pltpu.SemaphoreType.DMA,
          pltpu.SemaphoreType.DMA,
          pltpu.SemaphoreType.DMA,
          pltpu.SemaphoreType.DMA,
          pltpu.VMEM_SHARED((BP, D), jnp.float32),
          pltpu.VMEM_SHARED((BP, SEGW), jnp.float32),
      ],
  )
  def sc_kernel(h_hbm, b2d_hbm, e2d_hbm, w1_hbm,
                acc_out, seg_out,
                h_bufa, h_bufb, idx_bufa, idx_bufb, io_idx, ev_bufa, ev_bufb,
                w1_buf, sem_h0, sem_h1, sem_e0, sem_e1, sem_i0, sem_i1,
                sem_sa0, sem_sa1, sem_sb0, sem_sb1,
                acc_sh, es_sh):
    hb = [h_bufa, h_bufb]
    ixb = [idx_bufa, idx_bufb]
    evb = [ev_bufa, ev_bufb]
    sh = [sem_h0, sem_h1]
    se = [sem_e0, sem_e1]
    si = [sem_i0, sem_i1]
    ssa = [sem_sa0, sem_sa1]
    ssb = [sem_sb0, sem_sb1]
    cid = lax.axis_index("c")
    sid = lax.axis_index("s")
    wid = sid * NC + cid
    lanes = lax.iota(jnp.int32, 16)
    ecols = jnp.maximum(lanes, 1)
    zeros16i = jnp.zeros((16,), jnp.int32)
    zf16 = jnp.zeros((16,), jnp.float32)
    r0 = sid * RPS

    def set_io_idx(base):
      for c in range(SS // 16):
        io_idx[0, pl.ds(c * 16, 16)] = base + c * 16 + lanes

    def stream_base(rr):
      return r0 + jnp.minimum(rr * SS, RPS - SS)

    def _zrow(i, _):
      for kk in range(D // 16):
        h_bufa[i, pl.ds(kk * 16, 16)] = zf16
      return 0
    lax.fori_loop(0, SS, _zrow, 0)

    def _init(rr, _):
      base = stream_base(rr)
      set_io_idx(base)
      pltpu.sync_copy(h_bufa, acc_sh.at[io_idx.at[0]])
      pltpu.sync_copy(e2d_hbm.at[pl.ds(base, SS)], ev_bufa)
      pltpu.sync_copy(ev_bufa, es_sh.at[io_idx.at[0]])
      return 0
    lax.fori_loop(0, NRS + 1, _init, 0)
    pltpu.sync_copy(w1_hbm, w1_buf)
    w1v = [w1_buf[pl.ds(kk * 16, 16)] for kk in range(D // 16)]
    plsc.subcore_barrier()

    def issue_h(kc, jc, b):
      pltpu.async_copy(h_hbm.at[pl.ds(kc * C + jc * SS, SS)], hb[b], sh[b])

    def issue_ev(pc, jc, b):
      pltpu.async_copy(es_sh.at[ixb[pc].at[jc]], evb[b], se[b])

    def wait_h(b):
      pltpu.make_async_copy(h_hbm.at[pl.ds(0, SS)], hb[b], sh[b]).wait()

    def wait_ev(b):
      pltpu.make_async_copy(es_sh.at[ixb[0].at[0]], evb[b], se[b]).wait()

    def wait_scatters(b):
      pltpu.make_async_copy(hb[b], acc_sh.at[ixb[0].at[0]], ssa[b]).wait()
      pltpu.make_async_copy(evb[b], es_sh.at[ixb[0].at[0]], ssb[b]).wait()

    pltpu.async_copy(b2d_hbm.at[pl.ds(wid * NSUB, NSUB)], ixb[0], si[0])
    pltpu.make_async_copy(b2d_hbm.at[pl.ds(0, NSUB)], ixb[0], si[0]).wait()
    issue_h(wid, 0, 0)
    issue_ev(0, 0, 0)

    def compute_substage(cur, p, j):
      hc, ec = hb[cur], evb[cur]

      def group_body(g2, _):
        gbase = g2 * 16
        e16 = plsc.load_gather(ec, (gbase + lanes, ecols))
        for jj in range(16):
          row = [hc[gbase + jj, pl.ds(kk * 16, 16)] for kk in range(D // 16)]
          prt = row[0] * w1v[0]
          for kk in range(1, D // 16):
            prt = prt + row[kk] * w1v[kk]
          zj = jnp.sum(prt) + e16[jj]
          zj = jnp.where(zj >= 0, zj, zj * 0.01)
          av = jnp.exp(jnp.full((16,), zj))
          for kk in range(D // 16):
            hc[gbase + jj, pl.ds(kk * 16, 16)] = row[kk] * av
          ec[gbase + jj, :] = jnp.where(lanes == 0, av, zf16)
        return 0
      lax.fori_loop(0, GPS, group_body, 0)

    def pair_body(tp, _):
      for half in range(2):
        t = 2 * tp + half
        k = wid + t * NW
        p = half
        q = 1 - half

        @pl.when(k < NCHUNK)
        def _():
          for j in range(NSUB):
            cur = j % 2
            nxt = 1 - cur
            if half == 0 and j == 0:
              @pl.when(tp > 0)
              def _():
                wait_scatters(nxt)
            else:
              wait_scatters(nxt)
            if j == 0:
              @pl.when(k + NW < NCHUNK)
              def _():
                pltpu.async_copy(b2d_hbm.at[pl.ds((k + NW) * NSUB, NSUB)],
                                 ixb[q], si[q])
            if j < NSUB - 1:
              issue_h(k, j + 1, nxt)
              issue_ev(p, j + 1, nxt)
            else:
              @pl.when(k + NW < NCHUNK)
              def _():
                pltpu.make_async_copy(b2d_hbm.at[pl.ds(0, NSUB)],
                                      ixb[q], si[q]).wait()
                issue_h(k + NW, 0, nxt)
                issue_ev(q, 0, nxt)
            wait_h(cur)
            wait_ev(cur)
            compute_substage(cur, p, j)
            pltpu.async_copy(hb[cur], acc_sh.at[ixb[p].at[j]], ssa[cur])
            pltpu.async_copy(evb[cur], es_sh.at[ixb[p].at[j]], ssb[cur])
      return 0
    lax.fori_loop(0, (TPW + 1) // 2, pair_body, 0)
    wait_scatters(1)

    plsc.subcore_barrier()
    out_r0 = cid * BP + sid * RPS

    def _out(rr, _):
      base = stream_base(rr)
      set_io_idx(base)
      pltpu.sync_copy(acc_sh.at[io_idx.at[0]], h_bufa)
      pltpu.sync_copy(h_bufa, acc_out.at[pl.ds(out_r0 - r0 + base, SS)])
      pltpu.sync_copy(es_sh.at[io_idx.at[0]], ev_bufa)
      pltpu.sync_copy(ev_bufa, seg_out.at[pl.ds(out_r0 - r0 + base, SS)])
      return 0
    lax.fori_loop(0, NRS + 1, _out, 0)

  return sc_kernel(h_react, batch2d, e2d, w1)


def _e_kernel(henv_ref, w2_ref, b_ref, out_ref):
  ee = jnp.dot(henv_ref[...], w2_ref[...],
               preferred_element_type=jnp.float32) + b_ref[0, 0]
  col = lax.broadcasted_iota(jnp.int32, (B, SEGW), 1)
  out_ref[pl.ds(0, B), :] = jnp.where(col == 0, 0.0,
                                      jnp.broadcast_to(ee, (B, SEGW)))
  out_ref[pl.ds(B, BP - B), :] = jnp.zeros((BP - B, SEGW), jnp.float32)


def _mlp_kernel(acc_ref, seg_ref, henv_ref, fc1w_ref, fc1b_ref,
                fc2w_ref, fc2b_ref, out_ref):
  acc = acc_ref[pl.ds(0, B), :] + acc_ref[pl.ds(BP, B), :]
  seg = seg_ref[pl.ds(0, B), :] + seg_ref[pl.ds(BP, B), :]
  seg = seg[:, 0:1]
  seg = jnp.where(seg > 0, seg, 1.0)
  h_attn = acc / seg
  w_a = fc1w_ref[pl.ds(0, D), :]
  w_e = fc1w_ref[pl.ds(D, D), :]
  h1 = (jnp.dot(h_attn, w_a, preferred_element_type=jnp.float32)
        + jnp.dot(henv_ref[...], w_e, preferred_element_type=jnp.float32)
        + fc1b_ref[...])
  h1 = jnp.maximum(h1, 0.0)
  out_ref[...] = (jnp.dot(h1, fc2w_ref[...],
                          preferred_element_type=jnp.float32) + fc2b_ref[...])


def kernel(h_react, h_env, batch, attn_w, attn_b, fc1_w, fc1_b, fc2_w, fc2_b):
  w1 = attn_w[:D, 0]
  w2 = attn_w[D:, :]

  e2d = pl.pallas_call(
      _e_kernel,
      out_shape=jax.ShapeDtypeStruct((BP, SEGW), jnp.float32),
  )(h_env, w2, attn_b.reshape(1, 1))

  batch2d = batch.reshape(N // SS, SS)
  acc, seg = _sc_pass(h_react, batch2d, e2d, w1)

  out = pl.pallas_call(
      _mlp_kernel,
      out_shape=jax.ShapeDtypeStruct((B, 1), jnp.float32),
  )(acc, seg, h_env, fc1_w, fc1_b.reshape(1, 64), fc2_w, fc2_b.reshape(1, 1))
  return out
